# Initial kernel scaffold; baseline (speedup 1.0000x reference)
import jax
import jax.numpy as jnp
from jax.experimental import pallas as pl


def kernel(x, edge_index, W1l, b1, W1r, gamma, beta, W2l, b2, W2r, Wfc, bfc):
    raise NotImplementedError("write your pallas kernel here")


# speedup vs baseline: 4.3998x; 4.3998x over previous
import functools

import jax
import jax.numpy as jnp
from jax import lax
from jax.experimental import pallas as pl
from jax.experimental.pallas import tpu as pltpu
from jax.experimental.pallas import tpu_sc as plsc

_N = 10000
_E = 320000
_NC = 2
_NS = 16
_CHUNK = 80
_EPT = _E // (_NC * _NS)
_NCHUNK = _EPT // _CHUNK
_ZPT = _N // _CHUNK // _NS + 1
_OROWS = 632


def _zero_rows(ref, nrows, nlane16):
    def zrow(i, _):
        for j in range(nlane16):
            ref[i, pl.ds(j * 16, 16)] = jnp.zeros((16,), jnp.float32)
        return 0
    lax.fori_loop(0, nrows, zrow, 0)


def _agg_body(table, srcs, dsts, out, src_v, dst_v, rows_v, acc, sem):
    c = lax.axis_index("c")
    s = lax.axis_index("s")
    _zero_rows(rows_v, _CHUNK, 8)

    def zacc(i, _):
        start = jnp.minimum(i * _CHUNK, _N - _CHUNK)
        pltpu.sync_copy(rows_v, acc.at[pl.ds(start, _CHUNK)])
        return 0
    lax.fori_loop(s * _ZPT, (s + 1) * _ZPT, zacc, 0)
    plsc.subcore_barrier()

    base = (c * _NS + s) * _EPT

    def step(k, _):
        off = base + k * _CHUNK
        pltpu.sync_copy(srcs.at[pl.ds(off, _CHUNK)], src_v)
        pltpu.sync_copy(dsts.at[pl.ds(off, _CHUNK)], dst_v)
        pltpu.async_copy(table.at[src_v], rows_v, sem).wait()
        pltpu.sync_copy(rows_v, acc.at[dst_v], add=True)
        return 0
    lax.fori_loop(0, _NCHUNK, step, 0)
    plsc.subcore_barrier()

    start = jnp.minimum(s * _OROWS, _N - _OROWS)
    pltpu.sync_copy(acc.at[pl.ds(start, _OROWS)],
                    out.at[c, pl.ds(start, _OROWS)])


def _cnt_body(dsts, out, dst_v, ones_v, zeros_v, acc, sem):
    c = lax.axis_index("c")
    s = lax.axis_index("s")
    _zero_rows(zeros_v, _CHUNK, 8)

    def orow(i, _):
        for j in range(8):
            ones_v[i, pl.ds(j * 16, 16)] = jnp.ones((16,), jnp.float32)
        return 0
    lax.fori_loop(0, _CHUNK, orow, 0)

    def zacc(i, _):
        start = jnp.minimum(i * _CHUNK, _N - _CHUNK)
        pltpu.sync_copy(zeros_v, acc.at[pl.ds(start, _CHUNK)])
        return 0
    lax.fori_loop(s * _ZPT, (s + 1) * _ZPT, zacc, 0)
    plsc.subcore_barrier()

    base = (c * _NS + s) * _EPT

    def step(k, _):
        off = base + k * _CHUNK
        pltpu.sync_copy(dsts.at[pl.ds(off, _CHUNK)], dst_v)
        pltpu.sync_copy(ones_v, acc.at[dst_v], add=True)
        return 0
    lax.fori_loop(0, _NCHUNK, step, 0)
    plsc.subcore_barrier()

    start = jnp.minimum(s * _OROWS, _N - _OROWS)
    pltpu.sync_copy(acc.at[pl.ds(start, _OROWS)],
                    out.at[c, pl.ds(start, _OROWS)])


@functools.cache
def _get_mesh():
    return plsc.VectorSubcoreMesh(core_axis_name="c", subcore_axis_name="s",
                                  num_cores=_NC, num_subcores=_NS)


@functools.cache
def _get_agg():
    return pl.kernel(
        _agg_body, mesh=_get_mesh(),
        out_type=jax.ShapeDtypeStruct((_NC, _N, 128), jnp.float32),
        scratch_types=[pltpu.VMEM((_CHUNK,), jnp.int32),
                       pltpu.VMEM((_CHUNK,), jnp.int32),
                       pltpu.VMEM((_CHUNK, 128), jnp.float32),
                       pltpu.VMEM_SHARED((_N, 128), jnp.float32),
                       pltpu.SemaphoreType.DMA])


@functools.cache
def _get_cnt():
    return pl.kernel(
        _cnt_body, mesh=_get_mesh(),
        out_type=jax.ShapeDtypeStruct((_NC, _N, 128), jnp.float32),
        scratch_types=[pltpu.VMEM((_CHUNK,), jnp.int32),
                       pltpu.VMEM((_CHUNK, 128), jnp.float32),
                       pltpu.VMEM((_CHUNK, 128), jnp.float32),
                       pltpu.VMEM_SHARED((_N, 128), jnp.float32),
                       pltpu.SemaphoreType.DMA])


def _tc1_body(P, cnt, x, W1l, b1, W1r, gamma, beta, o):
    sums = P[0] + P[1]
    cnt1 = jnp.maximum((cnt[0] + cnt[1])[:, 0:1], 1.0)
    agg = sums / cnt1
    h = (jnp.dot(agg, W1l[...], preferred_element_type=jnp.float32)
         + jnp.dot(x[...], W1r[...], preferred_element_type=jnp.float32)
         + b1[...])
    nrm = jnp.sqrt(jnp.sum(h * h, axis=1, keepdims=True))
    h = h / jnp.maximum(nrm, 1e-12)
    h = jnp.maximum(h, 0.0)
    mu = jnp.mean(h, axis=0, keepdims=True)
    var = jnp.mean(jnp.square(h - mu), axis=0, keepdims=True)
    o[...] = (h - mu) * (gamma[...] / jnp.sqrt(var + 1e-5)) + beta[...]


def _tc2_body(P, cnt, h, W2l, b2, W2r, Wfc, bfc, o):
    sums = P[0] + P[1]
    cnt1 = jnp.maximum((cnt[0] + cnt[1])[:, 0:1], 1.0)
    agg = sums / cnt1
    h2 = (jnp.dot(agg, W2l[...], preferred_element_type=jnp.float32)
          + jnp.dot(h[...], W2r[...], preferred_element_type=jnp.float32)
          + b2[...])
    nrm = jnp.sqrt(jnp.sum(h2 * h2, axis=1, keepdims=True))
    h2 = h2 / jnp.maximum(nrm, 1e-12)
    o[...] = jnp.dot(h2, Wfc[...], preferred_element_type=jnp.float32) + bfc[...]


_tc1 = pl.pallas_call(
    _tc1_body, out_shape=jax.ShapeDtypeStruct((_N, 128), jnp.float32))
_tc2 = pl.pallas_call(
    _tc2_body, out_shape=jax.ShapeDtypeStruct((_N, 64), jnp.float32))


def kernel(x, edge_index, W1l, b1, W1r, gamma, beta, W2l, b2, W2r, Wfc, bfc):
    srcs = edge_index[0]
    dsts = edge_index[1]
    C2 = _get_cnt()(dsts)
    P1 = _get_agg()(x, srcs, dsts)
    h_bn = _tc1(P1, C2, x, W1l, b1.reshape(1, -1), W1r,
                gamma.reshape(1, -1), beta.reshape(1, -1))
    P2 = _get_agg()(h_bn, srcs, dsts)
    out = _tc2(P2, C2, h_bn, W2l, b2.reshape(1, -1), W2r,
               Wfc, bfc.reshape(1, -1))
    return out


# speedup vs baseline: 9.4895x; 2.1568x over previous
import functools

import jax
import jax.numpy as jnp
from jax import lax
from jax.experimental import pallas as pl
from jax.experimental.pallas import tpu as pltpu
from jax.experimental.pallas import tpu_sc as plsc

_N = 10000
_E = 320000
_NC = 2
_NS = 16
_CHUNK = 80
_EPT = _E // (_NC * _NS)
_NCHUNK = _EPT // _CHUNK
_ZPT = _N // _CHUNK // _NS + 1
_OROWS = 632


def _zero_rows(ref, nrows, nlane16):
    def zrow(i, _):
        for j in range(nlane16):
            ref[i, pl.ds(j * 16, 16)] = jnp.zeros((16,), jnp.float32)
        return 0
    lax.fori_loop(0, nrows, zrow, 0)


def _agg_body(table, srcs3, dsts3, out,
              src_all, d0, d1, buf0, buf1, acc,
              semg0, semg1, sems0, sems1, semd0, semd1):
    c = lax.axis_index("c")
    s = lax.axis_index("s")
    wid = c * _NS + s
    _zero_rows(buf0, _CHUNK, 8)
    _zero_rows(buf1, _CHUNK, 8)
    pltpu.sync_copy(srcs3.at[wid], src_all)
    pltpu.sync_copy(dsts3.at[wid, 0], d1)

    def zacc(i, _):
        start = jnp.minimum(i * _CHUNK, _N - _CHUNK)
        pltpu.sync_copy(buf0, acc.at[pl.ds(start, _CHUNK)])
        return 0
    lax.fori_loop(s * _ZPT, (s + 1) * _ZPT, zacc, 0)
    plsc.subcore_barrier()

    pltpu.async_copy(dsts3.at[wid, 0], d0, semd0)
    pltpu.async_copy(table.at[src_all.at[0]], buf0, semg0)
    pltpu.async_copy(buf1, acc.at[d1], sems1, add=True)
    klast = _NCHUNK - 1

    def pair(p, _):
        k0 = 2 * p
        k1 = k0 + 1
        pltpu.make_async_copy(buf1, acc.at[pl.ds(0, _CHUNK)], sems1).wait()
        pltpu.async_copy(dsts3.at[wid, k1], d1, semd1)
        pltpu.async_copy(table.at[src_all.at[k1]], buf1, semg1)
        pltpu.make_async_copy(table.at[src_all.at[k0]], buf0, semg0).wait()
        pltpu.make_async_copy(dsts3.at[wid, 0], d0, semd0).wait()
        pltpu.async_copy(buf0, acc.at[d0], sems0, add=True)
        pltpu.make_async_copy(buf0, acc.at[pl.ds(0, _CHUNK)], sems0).wait()
        pltpu.async_copy(dsts3.at[wid, jnp.minimum(k0 + 2, klast)], d0, semd0)
        pltpu.async_copy(table.at[src_all.at[jnp.minimum(k0 + 2, klast)]],
                         buf0, semg0)
        pltpu.make_async_copy(table.at[src_all.at[k1]], buf1, semg1).wait()
        pltpu.make_async_copy(dsts3.at[wid, 0], d1, semd1).wait()
        pltpu.async_copy(buf1, acc.at[d1], sems1, add=True)
        return 0
    lax.fori_loop(0, _NCHUNK // 2, pair, 0)
    pltpu.make_async_copy(buf1, acc.at[pl.ds(0, _CHUNK)], sems1).wait()
    pltpu.make_async_copy(table.at[src_all.at[klast]], buf0, semg0).wait()
    pltpu.make_async_copy(dsts3.at[wid, 0], d0, semd0).wait()
    pltpu.sync_copy(buf0, acc.at[d0], add=True)
    plsc.subcore_barrier()

    start = jnp.minimum(s * _OROWS, _N - _OROWS)
    pltpu.sync_copy(acc.at[pl.ds(start, _OROWS)],
                    out.at[c, pl.ds(start, _OROWS)])


def _cnt_body(dsts3, out, dst_all, ones_v, zeros_v, acc, sem):
    c = lax.axis_index("c")
    s = lax.axis_index("s")
    wid = c * _NS + s
    _zero_rows(zeros_v, _CHUNK, 8)

    def orow(i, _):
        for j in range(8):
            ones_v[i, pl.ds(j * 16, 16)] = jnp.ones((16,), jnp.float32)
        return 0
    lax.fori_loop(0, _CHUNK, orow, 0)
    pltpu.sync_copy(dsts3.at[wid], dst_all)

    def zacc(i, _):
        start = jnp.minimum(i * _CHUNK, _N - _CHUNK)
        pltpu.sync_copy(zeros_v, acc.at[pl.ds(start, _CHUNK)])
        return 0
    lax.fori_loop(s * _ZPT, (s + 1) * _ZPT, zacc, 0)
    plsc.subcore_barrier()

    pltpu.async_copy(ones_v, acc.at[dst_all.at[0]], sem, add=True)

    def step(k, _):
        pltpu.async_copy(ones_v, acc.at[dst_all.at[k]], sem, add=True)
        pltpu.make_async_copy(ones_v, acc.at[pl.ds(0, _CHUNK)], sem).wait()
        return 0
    lax.fori_loop(1, _NCHUNK, step, 0)
    pltpu.make_async_copy(ones_v, acc.at[pl.ds(0, _CHUNK)], sem).wait()
    plsc.subcore_barrier()

    start = jnp.minimum(s * _OROWS, _N - _OROWS)
    pltpu.sync_copy(acc.at[pl.ds(start, _OROWS)],
                    out.at[c, pl.ds(start, _OROWS)])


@functools.cache
def _get_mesh():
    return plsc.VectorSubcoreMesh(core_axis_name="c", subcore_axis_name="s",
                                  num_cores=_NC, num_subcores=_NS)


@functools.cache
def _get_agg():
    return pl.kernel(
        _agg_body, mesh=_get_mesh(),
        out_type=jax.ShapeDtypeStruct((_NC, _N, 128), jnp.float32),
        scratch_types=[pltpu.VMEM((_NCHUNK, _CHUNK), jnp.int32),
                       pltpu.VMEM((_CHUNK,), jnp.int32),
                       pltpu.VMEM((_CHUNK,), jnp.int32),
                       pltpu.VMEM((_CHUNK, 128), jnp.float32),
                       pltpu.VMEM((_CHUNK, 128), jnp.float32),
                       pltpu.VMEM_SHARED((_N, 128), jnp.float32),
                       pltpu.SemaphoreType.DMA,
                       pltpu.SemaphoreType.DMA,
                       pltpu.SemaphoreType.DMA,
                       pltpu.SemaphoreType.DMA,
                       pltpu.SemaphoreType.DMA,
                       pltpu.SemaphoreType.DMA])


@functools.cache
def _get_cnt():
    return pl.kernel(
        _cnt_body, mesh=_get_mesh(),
        out_type=jax.ShapeDtypeStruct((_NC, _N, 128), jnp.float32),
        scratch_types=[pltpu.VMEM((_NCHUNK, _CHUNK), jnp.int32),
                       pltpu.VMEM((_CHUNK, 128), jnp.float32),
                       pltpu.VMEM((_CHUNK, 128), jnp.float32),
                       pltpu.VMEM_SHARED((_N, 128), jnp.float32),
                       pltpu.SemaphoreType.DMA])


def _tc1_body(P, cnt, x, W1l, b1, W1r, gamma, beta, o):
    sums = P[0] + P[1]
    cnt1 = jnp.maximum((cnt[0] + cnt[1])[:, 0:1], 1.0)
    agg = sums / cnt1
    h = (jnp.dot(agg, W1l[...], preferred_element_type=jnp.float32)
         + jnp.dot(x[...], W1r[...], preferred_element_type=jnp.float32)
         + b1[...])
    nrm = jnp.sqrt(jnp.sum(h * h, axis=1, keepdims=True))
    h = h / jnp.maximum(nrm, 1e-12)
    h = jnp.maximum(h, 0.0)
    mu = jnp.mean(h, axis=0, keepdims=True)
    var = jnp.mean(jnp.square(h - mu), axis=0, keepdims=True)
    o[...] = (h - mu) * (gamma[...] / jnp.sqrt(var + 1e-5)) + beta[...]


def _tc2_body(P, cnt, h, W2l, b2, W2r, Wfc, bfc, o):
    sums = P[0] + P[1]
    cnt1 = jnp.maximum((cnt[0] + cnt[1])[:, 0:1], 1.0)
    agg = sums / cnt1
    h2 = (jnp.dot(agg, W2l[...], preferred_element_type=jnp.float32)
          + jnp.dot(h[...], W2r[...], preferred_element_type=jnp.float32)
          + b2[...])
    nrm = jnp.sqrt(jnp.sum(h2 * h2, axis=1, keepdims=True))
    h2 = h2 / jnp.maximum(nrm, 1e-12)
    o[...] = jnp.dot(h2, Wfc[...], preferred_element_type=jnp.float32) + bfc[...]


_tc1 = pl.pallas_call(
    _tc1_body, out_shape=jax.ShapeDtypeStruct((_N, 128), jnp.float32))
_tc2 = pl.pallas_call(
    _tc2_body, out_shape=jax.ShapeDtypeStruct((_N, 64), jnp.float32))


def kernel(x, edge_index, W1l, b1, W1r, gamma, beta, W2l, b2, W2r, Wfc, bfc):
    srcs3 = edge_index[0].reshape(_NC * _NS, _NCHUNK, _CHUNK)
    dsts3 = edge_index[1].reshape(_NC * _NS, _NCHUNK, _CHUNK)
    C2 = _get_cnt()(dsts3)
    P1 = _get_agg()(x, srcs3, dsts3)
    h_bn = _tc1(P1, C2, x, W1l, b1.reshape(1, -1), W1r,
                gamma.reshape(1, -1), beta.reshape(1, -1))
    P2 = _get_agg()(h_bn, srcs3, dsts3)
    out = _tc2(P2, C2, h_bn, W2l, b2.reshape(1, -1), W2r,
               Wfc, bfc.reshape(1, -1))
    return out


# speedup vs baseline: 9.4907x; 1.0001x over previous
import functools

import jax
import jax.numpy as jnp
from jax import lax
from jax.experimental import pallas as pl
from jax.experimental.pallas import tpu as pltpu
from jax.experimental.pallas import tpu_sc as plsc

_N = 10000
_E = 320000
_NC = 2
_NS = 16
_CHUNK = 80
_EPT = _E // (_NC * _NS)
_NCHUNK = _EPT // _CHUNK
_ZPT = _N // _CHUNK // _NS + 1
_OROWS = 632


def _zero_rows(ref, nrows, nlane16):
    def zrow(i, _):
        for j in range(nlane16):
            ref[i, pl.ds(j * 16, 16)] = jnp.zeros((16,), jnp.float32)
        return 0
    lax.fori_loop(0, nrows, zrow, 0)


def _agg_body(table, srcs3, dsts3, out,
              src_all, d0, d1, buf0, buf1, acc,
              semg0, semg1, sems0, sems1, semd0, semd1):
    c = lax.axis_index("c")
    s = lax.axis_index("s")
    wid = c * _NS + s
    _zero_rows(buf0, _CHUNK, 8)
    _zero_rows(buf1, _CHUNK, 8)
    pltpu.sync_copy(srcs3.at[wid], src_all)
    pltpu.sync_copy(dsts3.at[wid, 0], d1)

    def zacc(i, _):
        start = jnp.minimum(i * _CHUNK, _N - _CHUNK)
        pltpu.sync_copy(buf0, acc.at[pl.ds(start, _CHUNK)])
        return 0
    lax.fori_loop(s * _ZPT, (s + 1) * _ZPT, zacc, 0)
    plsc.subcore_barrier()

    pltpu.async_copy(dsts3.at[wid, 0], d0, semd0)
    pltpu.async_copy(table.at[src_all.at[0]], buf0, semg0)
    pltpu.async_copy(buf1, acc.at[d1], sems1, add=True)
    klast = _NCHUNK - 1

    def pair(p, _):
        k0 = 2 * p
        k1 = k0 + 1
        pltpu.make_async_copy(buf1, acc.at[pl.ds(0, _CHUNK)], sems1).wait()
        pltpu.async_copy(dsts3.at[wid, k1], d1, semd1)
        pltpu.async_copy(table.at[src_all.at[k1]], buf1, semg1)
        pltpu.make_async_copy(table.at[src_all.at[k0]], buf0, semg0).wait()
        pltpu.make_async_copy(dsts3.at[wid, 0], d0, semd0).wait()
        pltpu.async_copy(buf0, acc.at[d0], sems0, add=True)
        pltpu.make_async_copy(buf0, acc.at[pl.ds(0, _CHUNK)], sems0).wait()
        pltpu.async_copy(dsts3.at[wid, jnp.minimum(k0 + 2, klast)], d0, semd0)
        pltpu.async_copy(table.at[src_all.at[jnp.minimum(k0 + 2, klast)]],
                         buf0, semg0)
        pltpu.make_async_copy(table.at[src_all.at[k1]], buf1, semg1).wait()
        pltpu.make_async_copy(dsts3.at[wid, 0], d1, semd1).wait()
        pltpu.async_copy(buf1, acc.at[d1], sems1, add=True)
        return 0
    lax.fori_loop(0, _NCHUNK // 2, pair, 0)
    pltpu.make_async_copy(buf1, acc.at[pl.ds(0, _CHUNK)], sems1).wait()
    pltpu.make_async_copy(table.at[src_all.at[klast]], buf0, semg0).wait()
    pltpu.make_async_copy(dsts3.at[wid, 0], d0, semd0).wait()
    pltpu.sync_copy(buf0, acc.at[d0], add=True)
    plsc.subcore_barrier()

    start = jnp.minimum(s * _OROWS, _N - _OROWS)
    pltpu.sync_copy(acc.at[pl.ds(start, _OROWS)],
                    out.at[c, pl.ds(start, _OROWS)])


def _cnt_body(dsts3, out, dst_all, ones_v, zeros_v, acc, sem):
    c = lax.axis_index("c")
    s = lax.axis_index("s")
    wid = c * _NS + s
    _zero_rows(zeros_v, _CHUNK, 8)

    def orow(i, _):
        for j in range(8):
            ones_v[i, pl.ds(j * 16, 16)] = jnp.ones((16,), jnp.float32)
        return 0
    lax.fori_loop(0, _CHUNK, orow, 0)
    pltpu.sync_copy(dsts3.at[wid], dst_all)

    def zacc(i, _):
        start = jnp.minimum(i * _CHUNK, _N - _CHUNK)
        pltpu.sync_copy(zeros_v, acc.at[pl.ds(start, _CHUNK)])
        return 0
    lax.fori_loop(s * _ZPT, (s + 1) * _ZPT, zacc, 0)
    plsc.subcore_barrier()

    for k0 in range(4):
        pltpu.async_copy(ones_v, acc.at[dst_all.at[k0]], sem, add=True)

    def step(k, _):
        pltpu.async_copy(ones_v, acc.at[dst_all.at[k]], sem, add=True)
        pltpu.make_async_copy(ones_v, acc.at[pl.ds(0, _CHUNK)], sem).wait()
        return 0
    lax.fori_loop(4, _NCHUNK, step, 0)
    for k0 in range(4):
        pltpu.make_async_copy(ones_v, acc.at[pl.ds(0, _CHUNK)], sem).wait()
    plsc.subcore_barrier()

    start = jnp.minimum(s * _OROWS, _N - _OROWS)
    pltpu.sync_copy(acc.at[pl.ds(start, _OROWS)],
                    out.at[c, pl.ds(start, _OROWS)])


@functools.cache
def _get_mesh():
    return plsc.VectorSubcoreMesh(core_axis_name="c", subcore_axis_name="s",
                                  num_cores=_NC, num_subcores=_NS)


@functools.cache
def _get_agg():
    return pl.kernel(
        _agg_body, mesh=_get_mesh(),
        out_type=jax.ShapeDtypeStruct((_NC, _N, 128), jnp.float32),
        scratch_types=[pltpu.VMEM((_NCHUNK, _CHUNK), jnp.int32),
                       pltpu.VMEM((_CHUNK,), jnp.int32),
                       pltpu.VMEM((_CHUNK,), jnp.int32),
                       pltpu.VMEM((_CHUNK, 128), jnp.float32),
                       pltpu.VMEM((_CHUNK, 128), jnp.float32),
                       pltpu.VMEM_SHARED((_N, 128), jnp.float32),
                       pltpu.SemaphoreType.DMA,
                       pltpu.SemaphoreType.DMA,
                       pltpu.SemaphoreType.DMA,
                       pltpu.SemaphoreType.DMA,
                       pltpu.SemaphoreType.DMA,
                       pltpu.SemaphoreType.DMA])


@functools.cache
def _get_cnt():
    return pl.kernel(
        _cnt_body, mesh=_get_mesh(),
        out_type=jax.ShapeDtypeStruct((_NC, _N, 128), jnp.float32),
        scratch_types=[pltpu.VMEM((_NCHUNK, _CHUNK), jnp.int32),
                       pltpu.VMEM((_CHUNK, 128), jnp.float32),
                       pltpu.VMEM((_CHUNK, 128), jnp.float32),
                       pltpu.VMEM_SHARED((_N, 128), jnp.float32),
                       pltpu.SemaphoreType.DMA])


def _tc1_body(P, cnt, x, W1l, b1, W1r, gamma, beta, o):
    sums = P[0] + P[1]
    cnt1 = jnp.maximum((cnt[0] + cnt[1])[:, 0:1], 1.0)
    agg = sums / cnt1
    h = (jnp.dot(agg, W1l[...], preferred_element_type=jnp.float32)
         + jnp.dot(x[...], W1r[...], preferred_element_type=jnp.float32)
         + b1[...])
    nrm = jnp.sqrt(jnp.sum(h * h, axis=1, keepdims=True))
    h = h / jnp.maximum(nrm, 1e-12)
    h = jnp.maximum(h, 0.0)
    mu = jnp.mean(h, axis=0, keepdims=True)
    var = jnp.mean(jnp.square(h - mu), axis=0, keepdims=True)
    o[...] = (h - mu) * (gamma[...] / jnp.sqrt(var + 1e-5)) + beta[...]


def _tc2_body(P, cnt, h, W2l, b2, W2r, Wfc, bfc, o):
    sums = P[0] + P[1]
    cnt1 = jnp.maximum((cnt[0] + cnt[1])[:, 0:1], 1.0)
    agg = sums / cnt1
    h2 = (jnp.dot(agg, W2l[...], preferred_element_type=jnp.float32)
          + jnp.dot(h[...], W2r[...], preferred_element_type=jnp.float32)
          + b2[...])
    nrm = jnp.sqrt(jnp.sum(h2 * h2, axis=1, keepdims=True))
    h2 = h2 / jnp.maximum(nrm, 1e-12)
    o[...] = jnp.dot(h2, Wfc[...], preferred_element_type=jnp.float32) + bfc[...]


_tc1 = pl.pallas_call(
    _tc1_body, out_shape=jax.ShapeDtypeStruct((_N, 128), jnp.float32))
_tc2 = pl.pallas_call(
    _tc2_body, out_shape=jax.ShapeDtypeStruct((_N, 64), jnp.float32))


def kernel(x, edge_index, W1l, b1, W1r, gamma, beta, W2l, b2, W2r, Wfc, bfc):
    srcs3 = edge_index[0].reshape(_NC * _NS, _NCHUNK, _CHUNK)
    dsts3 = edge_index[1].reshape(_NC * _NS, _NCHUNK, _CHUNK)
    C2 = _get_cnt()(dsts3)
    P1 = _get_agg()(x, srcs3, dsts3)
    h_bn = _tc1(P1, C2, x, W1l, b1.reshape(1, -1), W1r,
                gamma.reshape(1, -1), beta.reshape(1, -1))
    P2 = _get_agg()(h_bn, srcs3, dsts3)
    out = _tc2(P2, C2, h_bn, W2l, b2.reshape(1, -1), W2r,
               Wfc, bfc.reshape(1, -1))
    return out


# speedup vs baseline: 10.4880x; 1.1051x over previous
import functools

import jax
import jax.numpy as jnp
from jax import lax
from jax.experimental import pallas as pl
from jax.experimental.pallas import tpu as pltpu
from jax.experimental.pallas import tpu_sc as plsc

_N = 10000
_E = 320000
_NC = 2
_NS = 16
_CHUNK = 80
_EPT = _E // (_NC * _NS)
_NCHUNK = _EPT // _CHUNK
_ZPT = _N // _CHUNK // _NS + 1
_OROWS = 632


def _zero_rows(ref, nrows, nlane16):
    def zrow(i, _):
        for j in range(nlane16):
            ref[i, pl.ds(j * 16, 16)] = jnp.zeros((16,), jnp.float32)
        return 0
    lax.fori_loop(0, nrows, zrow, 0)


def _agg_body(table, srcs3, dsts3, out,
              si, d, b, acc, semi, semd, semg, sems):
    c = lax.axis_index("c")
    s = lax.axis_index("s")
    wid = c * _NS + s
    for j in range(3):
        _zero_rows(b[j], _CHUNK, 8)
    pltpu.sync_copy(dsts3.at[wid, 0], d[2])

    def zacc(i, _):
        start = jnp.minimum(i * _CHUNK, _N - _CHUNK)
        pltpu.sync_copy(b[0], acc.at[pl.ds(start, _CHUNK)])
        return 0
    lax.fori_loop(s * _ZPT, (s + 1) * _ZPT, zacc, 0)
    plsc.subcore_barrier()

    for j in range(2):
        pltpu.async_copy(srcs3.at[wid, j], si[j], semi[j])
        pltpu.async_copy(dsts3.at[wid, j], d[j], semd[j])
    pltpu.async_copy(srcs3.at[wid, 2], si[2], semi[2])
    for j in range(2):
        pltpu.make_async_copy(srcs3.at[wid, 0], si[j], semi[j]).wait()
        pltpu.async_copy(table.at[si[j]], b[j], semg[j])
    pltpu.async_copy(b[2], acc.at[d[2]], sems[2], add=True)

    def _wait(sem, dst):
        pltpu.make_async_copy(dsts3.at[wid, 0], dst, sem).wait()

    def step(t, _):
        for j in range(3):
            k = 3 * t + j
            jn = (j + 2) % 3
            pltpu.make_async_copy(table.at[si[j]], b[j], semg[j]).wait()
            _wait(semd[j], d[j])
            pltpu.async_copy(b[j], acc.at[d[j]], sems[j], add=True)
            pltpu.make_async_copy(b[jn], acc.at[pl.ds(0, _CHUNK)],
                                  sems[jn]).wait()
            pltpu.async_copy(dsts3.at[wid, k + 2], d[jn], semd[jn])
            pltpu.async_copy(srcs3.at[wid, jnp.minimum(k + 3, _NCHUNK - 1)],
                             si[j], semi[j])
            _wait(semi[jn], si[jn])
            pltpu.async_copy(table.at[si[jn]], b[jn], semg[jn])
        return 0
    lax.fori_loop(0, (_NCHUNK - 2) // 3, step, 0)
    for j in range(2):
        pltpu.make_async_copy(table.at[si[j]], b[j], semg[j]).wait()
        _wait(semd[j], d[j])
        pltpu.async_copy(b[j], acc.at[d[j]], sems[j], add=True)
        pltpu.make_async_copy(b[(j + 2) % 3], acc.at[pl.ds(0, _CHUNK)],
                              sems[(j + 2) % 3]).wait()
    pltpu.make_async_copy(b[1], acc.at[pl.ds(0, _CHUNK)], sems[1]).wait()
    _wait(semi[2], si[2])
    plsc.subcore_barrier()

    start = jnp.minimum(s * _OROWS, _N - _OROWS)
    pltpu.sync_copy(acc.at[pl.ds(start, _OROWS)],
                    out.at[c, pl.ds(start, _OROWS)])


def _cnt_body(dsts3, out, dst_all, ones_v, zeros_v, acc, sem):
    c = lax.axis_index("c")
    s = lax.axis_index("s")
    wid = c * _NS + s
    _zero_rows(zeros_v, _CHUNK, 8)

    def orow(i, _):
        for j in range(8):
            ones_v[i, pl.ds(j * 16, 16)] = jnp.ones((16,), jnp.float32)
        return 0
    lax.fori_loop(0, _CHUNK, orow, 0)
    pltpu.sync_copy(dsts3.at[wid], dst_all)

    def zacc(i, _):
        start = jnp.minimum(i * _CHUNK, _N - _CHUNK)
        pltpu.sync_copy(zeros_v, acc.at[pl.ds(start, _CHUNK)])
        return 0
    lax.fori_loop(s * _ZPT, (s + 1) * _ZPT, zacc, 0)
    plsc.subcore_barrier()

    for k0 in range(4):
        pltpu.async_copy(ones_v, acc.at[dst_all.at[k0]], sem, add=True)

    def step(k, _):
        pltpu.async_copy(ones_v, acc.at[dst_all.at[k]], sem, add=True)
        pltpu.make_async_copy(ones_v, acc.at[pl.ds(0, _CHUNK)], sem).wait()
        return 0
    lax.fori_loop(4, _NCHUNK, step, 0)
    for k0 in range(4):
        pltpu.make_async_copy(ones_v, acc.at[pl.ds(0, _CHUNK)], sem).wait()
    plsc.subcore_barrier()

    start = jnp.minimum(s * _OROWS, _N - _OROWS)
    pltpu.sync_copy(acc.at[pl.ds(start, _OROWS)],
                    out.at[c, pl.ds(start, _OROWS)])


@functools.cache
def _get_mesh():
    return plsc.VectorSubcoreMesh(core_axis_name="c", subcore_axis_name="s",
                                  num_cores=_NC, num_subcores=_NS)


@functools.cache
def _get_agg():
    return pl.kernel(
        _agg_body, mesh=_get_mesh(),
        out_type=jax.ShapeDtypeStruct((_NC, _N, 128), jnp.float32),
        scratch_types=[[pltpu.VMEM((_CHUNK,), jnp.int32) for _ in range(3)],
                       [pltpu.VMEM((_CHUNK,), jnp.int32) for _ in range(3)],
                       [pltpu.VMEM((_CHUNK, 128), jnp.float32)
                        for _ in range(3)],
                       pltpu.VMEM_SHARED((_N, 128), jnp.float32),
                       [pltpu.SemaphoreType.DMA for _ in range(3)],
                       [pltpu.SemaphoreType.DMA for _ in range(3)],
                       [pltpu.SemaphoreType.DMA for _ in range(3)],
                       [pltpu.SemaphoreType.DMA for _ in range(3)]])


@functools.cache
def _get_cnt():
    return pl.kernel(
        _cnt_body, mesh=_get_mesh(),
        out_type=jax.ShapeDtypeStruct((_NC, _N, 128), jnp.float32),
        scratch_types=[pltpu.VMEM((_NCHUNK, _CHUNK), jnp.int32),
                       pltpu.VMEM((_CHUNK, 128), jnp.float32),
                       pltpu.VMEM((_CHUNK, 128), jnp.float32),
                       pltpu.VMEM_SHARED((_N, 128), jnp.float32),
                       pltpu.SemaphoreType.DMA])


def _tc1_body(P, cnt, x, W1l, b1, W1r, gamma, beta, o):
    sums = P[0] + P[1]
    cnt1 = jnp.maximum((cnt[0] + cnt[1])[:, 0:1], 1.0)
    agg = sums / cnt1
    h = (jnp.dot(agg, W1l[...], preferred_element_type=jnp.float32)
         + jnp.dot(x[...], W1r[...], preferred_element_type=jnp.float32)
         + b1[...])
    nrm = jnp.sqrt(jnp.sum(h * h, axis=1, keepdims=True))
    h = h / jnp.maximum(nrm, 1e-12)
    h = jnp.maximum(h, 0.0)
    mu = jnp.mean(h, axis=0, keepdims=True)
    var = jnp.mean(jnp.square(h - mu), axis=0, keepdims=True)
    o[...] = (h - mu) * (gamma[...] / jnp.sqrt(var + 1e-5)) + beta[...]


def _tc2_body(P, cnt, h, W2l, b2, W2r, Wfc, bfc, o):
    sums = P[0] + P[1]
    cnt1 = jnp.maximum((cnt[0] + cnt[1])[:, 0:1], 1.0)
    agg = sums / cnt1
    h2 = (jnp.dot(agg, W2l[...], preferred_element_type=jnp.float32)
          + jnp.dot(h[...], W2r[...], preferred_element_type=jnp.float32)
          + b2[...])
    nrm = jnp.sqrt(jnp.sum(h2 * h2, axis=1, keepdims=True))
    h2 = h2 / jnp.maximum(nrm, 1e-12)
    o[...] = jnp.dot(h2, Wfc[...], preferred_element_type=jnp.float32) + bfc[...]


_tc1 = pl.pallas_call(
    _tc1_body, out_shape=jax.ShapeDtypeStruct((_N, 128), jnp.float32))
_tc2 = pl.pallas_call(
    _tc2_body, out_shape=jax.ShapeDtypeStruct((_N, 64), jnp.float32))


def kernel(x, edge_index, W1l, b1, W1r, gamma, beta, W2l, b2, W2r, Wfc, bfc):
    srcs3 = edge_index[0].reshape(_NC * _NS, _NCHUNK, _CHUNK)
    dsts3 = edge_index[1].reshape(_NC * _NS, _NCHUNK, _CHUNK)
    C2 = _get_cnt()(dsts3)
    P1 = _get_agg()(x, srcs3, dsts3)
    h_bn = _tc1(P1, C2, x, W1l, b1.reshape(1, -1), W1r,
                gamma.reshape(1, -1), beta.reshape(1, -1))
    P2 = _get_agg()(h_bn, srcs3, dsts3)
    out = _tc2(P2, C2, h_bn, W2l, b2.reshape(1, -1), W2r,
               Wfc, bfc.reshape(1, -1))
    return out
